# baseline (speedup 1.0000x reference)
import functools

import jax
import jax.numpy as jnp
from jax import lax
from jax.experimental import pallas as pl
from jax.experimental.pallas import tpu as pltpu

N_ = 10000
E_ = 160000
T_ = 320000
G_ = 64
H_ = 128
INT_ = 64
NR_ = 6
NS_ = 3
CUTOFF = 5.0

BE = 640
BT = 640
BN = 400

_F32 = jnp.float32


def _swish(x):
    return x / (1.0 + jnp.exp(-x))


def _dot(a, b):
    return jnp.dot(a, b, preferred_element_type=_F32)


def _dist_emb(d):
    x = d / CUTOFF
    x2 = x * x
    x4 = x2 * x2
    x5 = x4 * x
    x6 = x5 * x
    env = 1.0 / x - 21.0 * x4 + 35.0 * x5 - 15.0 * x6
    n = (lax.broadcasted_iota(jnp.int32, (1, NR_), 1) + 1).astype(_F32)
    return env * jnp.sin(n * jnp.pi * x)


def _gather_rows(table, idx):
    return jnp.take(table, idx, axis=0)


def _scatter_add(src, idx, size):
    return jax.ops.segment_sum(src, idx, num_segments=size)


def _counts(idx, size):
    c = jax.ops.segment_sum(jnp.ones(idx.shape, _F32), idx, num_segments=size)
    return jnp.tile(c[:, None], (1, 16))


def _full(shape):
    return pl.BlockSpec(shape, lambda b: (0,) * len(shape))


def _rows(w):
    return pl.BlockSpec((BE, w), lambda b: (b, 0))


def _init_body(dist_ref, xi_ref, xj_ref, w0_ref, b0_ref, wl1_ref, wl2_ref,
               wl3_ref, bl_ref, rbf1w_ref, rbf0_out, e1_out, e2_out):
    rbf = _dist_emb(dist_ref[...])
    r0 = _swish(_dot(rbf, w0_ref[...]) + b0_ref[...])
    e1 = _swish(_dot(xi_ref[...], wl1_ref[...]) + _dot(xj_ref[...], wl2_ref[...])
                + _dot(r0, wl3_ref[...]) + bl_ref[...])
    e2 = _dot(rbf, rbf1w_ref[...]) * e1
    rbf0_out[...] = jnp.concatenate(
        [rbf, jnp.zeros((rbf.shape[0], 16 - NR_), _F32)], axis=1)
    e1_out[...] = e1
    e2_out[...] = e2


def _call_init(dist2, xi, xj, p0):
    wl = p0['lin_w']
    return pl.pallas_call(
        _init_body,
        grid=(E_ // BE,),
        in_specs=[_rows(1), _rows(H_), _rows(H_), _full((NR_, H_)),
                  _full((1, H_)), _full((H_, H_)), _full((H_, H_)),
                  _full((H_, H_)), _full((1, H_)), _full((NR_, H_))],
        out_specs=[_rows(16), _rows(H_), _rows(H_)],
        out_shape=[jax.ShapeDtypeStruct((E_, 16), _F32),
                   jax.ShapeDtypeStruct((E_, H_), _F32),
                   jax.ShapeDtypeStruct((E_, H_), _F32)],
    )(dist2, xi, xj, p0['rbf0_w'], p0['rbf0_b'].reshape(1, H_),
      wl[:H_], wl[H_:2 * H_], wl[2 * H_:], p0['lin_b'].reshape(1, H_),
      p0['rbf1_w'])


def _edge1_body(e1_ref, rbf0_ref, jiw, jib, kjw, kjb, rbf1, rbf2, down,
                xji_out, xkd_out):
    x1 = e1_ref[...]
    rbf6 = rbf0_ref[:, :NR_]
    xji_out[...] = _swish(_dot(x1, jiw[...]) + jib[...])
    x_kj = _swish(_dot(x1, kjw[...]) + kjb[...])
    x_kj = x_kj * _dot(_dot(rbf6, rbf1[...]), rbf2[...])
    xkd_out[...] = _swish(_dot(x_kj, down[...]))


def _call_edge1(e1, rbf0p, lp):
    return pl.pallas_call(
        _edge1_body,
        grid=(E_ // BE,),
        in_specs=[_rows(H_), _rows(16), _full((H_, H_)), _full((1, H_)),
                  _full((H_, H_)), _full((1, H_)), _full((NR_, 8)),
                  _full((8, H_)), _full((H_, INT_))],
        out_specs=[_rows(H_), _rows(INT_)],
        out_shape=[jax.ShapeDtypeStruct((E_, H_), _F32),
                   jax.ShapeDtypeStruct((E_, INT_), _F32)],
    )(e1, rbf0p, lp['ji_w'], lp['ji_b'].reshape(1, H_), lp['kj_w'],
      lp['kj_b'].reshape(1, H_), lp['rbf1'], lp['rbf2'], lp['down'])


def _trip_body(g_ref, rbfk_ref, ang_ref, tor_ref, sbf1, sbf2, t1, t2, y_out):
    rbf6 = rbfk_ref[:, :NR_]
    l = lax.broadcasted_iota(jnp.int32, (1, NS_), 1).astype(_F32)
    cbf = jnp.cos(l * ang_ref[...])
    tbf = jnp.cos(l * tor_ref[...])
    sbf = jnp.concatenate([rbf6 * cbf[:, s:s + 1] for s in range(NS_)], axis=1)
    tin = jnp.concatenate(
        [rbf6 * cbf[:, s:s + 1] * tbf[:, k:k + 1]
         for s in range(NS_) for k in range(NS_)], axis=1)
    sb = _dot(_dot(sbf, sbf1[...]), sbf2[...])
    tb = _dot(_dot(tin, t1[...]), t2[...])
    y_out[...] = g_ref[...] * sb * tb


def _call_trip(g, rbfk, ang2, tor2, lp):
    tr = lambda w: pl.BlockSpec((BT, w), lambda b: (b, 0))
    return pl.pallas_call(
        _trip_body,
        grid=(T_ // BT,),
        in_specs=[tr(INT_), tr(16), tr(1), tr(1), _full((NS_ * NR_, 8)),
                  _full((8, INT_)), _full((NS_ * NS_ * NR_, 8)),
                  _full((8, INT_))],
        out_specs=tr(INT_),
        out_shape=jax.ShapeDtypeStruct((T_, INT_), _F32),
    )(g, rbfk, ang2, tor2, lp['sbf1'], lp['sbf2'], lp['t1'], lp['t2'])


def _edge2_body(ssum_ref, cnt_ref, xji_ref, x1_ref, rbf0_ref, up,
                bw1, bb1, bw2, bb2, linw, linb,
                aw1, ab1, aw2, ab2, aw3, ab3, aw4, ab4, rbfw,
                e1_out, e2_out):
    cnt = jnp.maximum(cnt_ref[:, :1], 1.0)
    xkj = _swish(_dot(ssum_ref[...] / cnt, up[...]))
    e1 = xji_ref[...] + xkj
    e1 = e1 + _swish(_dot(_swish(_dot(e1, bw1[...]) + bb1[...]), bw2[...]) + bb2[...])
    e1 = _swish(_dot(e1, linw[...]) + linb[...]) + x1_ref[...]
    e1 = e1 + _swish(_dot(_swish(_dot(e1, aw1[...]) + ab1[...]), aw2[...]) + ab2[...])
    e1 = e1 + _swish(_dot(_swish(_dot(e1, aw3[...]) + ab3[...]), aw4[...]) + ab4[...])
    e1_out[...] = e1
    e2_out[...] = _dot(rbf0_ref[:, :NR_], rbfw[...]) * e1


def _call_edge2(ssum, cnt_e, xji, e1_old, rbf0p, lp):
    (bw1, bb1, bw2, bb2), = lp['before']
    (aw1, ab1, aw2, ab2), (aw3, ab3, aw4, ab4) = lp['after']
    r1 = lambda: _full((1, H_))
    hh = lambda: _full((H_, H_))
    return pl.pallas_call(
        _edge2_body,
        grid=(E_ // BE,),
        in_specs=[_rows(INT_), _rows(16), _rows(H_), _rows(H_), _rows(16),
                  _full((INT_, H_)), hh(), r1(), hh(), r1(), hh(), r1(),
                  hh(), r1(), hh(), r1(), hh(), r1(), hh(), r1(),
                  _full((NR_, H_))],
        out_specs=[_rows(H_), _rows(H_)],
        out_shape=[jax.ShapeDtypeStruct((E_, H_), _F32),
                   jax.ShapeDtypeStruct((E_, H_), _F32)],
    )(ssum, cnt_e, xji, e1_old, rbf0p, lp['up'],
      bw1, bb1.reshape(1, H_), bw2, bb2.reshape(1, H_),
      lp['lin_w'], lp['lin_b'].reshape(1, H_),
      aw1, ab1.reshape(1, H_), aw2, ab2.reshape(1, H_),
      aw3, ab3.reshape(1, H_), aw4, ab4.reshape(1, H_), lp['rbf'])


def _vfn_body(sa_ref, sb_ref, cnt_ref, batch_ref, upw, upb, l1w, l1b, l2w,
              l2b, l3w, l3b, outw, u_out):
    @pl.when(pl.program_id(0) == 0)
    def _():
        u_out[...] = jnp.zeros_like(u_out)

    cnt = jnp.maximum(cnt_ref[:, :1], 1.0)
    v = (sa_ref[...] + sb_ref[...]) / cnt
    v = _dot(v, upw[...]) + upb[...]
    v = _swish(_dot(v, l1w[...]) + l1b[...])
    v = _swish(_dot(v, l2w[...]) + l2b[...])
    v = _swish(_dot(v, l3w[...]) + l3b[...])
    v = _dot(v, outw[...])
    b = batch_ref[0]
    oh = (lax.broadcasted_iota(jnp.int32, (G_, BN), 0)
          == jnp.broadcast_to(b, (G_, BN))).astype(_F32)
    u_out[...] += jnp.broadcast_to(_dot(oh, v), (G_, H_))


def _call_vfn(sa, sb, cnt_n, batch3, vp):
    nr = lambda w: pl.BlockSpec((BN, w), lambda b: (b, 0))
    (l1w, l1b), (l2w, l2b), (l3w, l3b) = vp['lins']
    oo = 256
    return pl.pallas_call(
        _vfn_body,
        grid=(N_ // BN,),
        in_specs=[nr(H_), nr(H_), nr(16),
                  pl.BlockSpec((1, 1, BN), lambda b: (b, 0, 0)),
                  _full((H_, oo)), _full((1, oo)), _full((oo, oo)),
                  _full((1, oo)), _full((oo, oo)), _full((1, oo)),
                  _full((oo, oo)), _full((1, oo)), _full((oo, 1))],
        out_specs=pl.BlockSpec((G_, H_), lambda b: (0, 0)),
        out_shape=jax.ShapeDtypeStruct((G_, H_), _F32),
    )(sa, sb, cnt_n, batch3, vp['up_w'], vp['up_b'].reshape(1, oo),
      l1w, l1b.reshape(1, oo), l2w, l2b.reshape(1, oo),
      l3w, l3b.reshape(1, oo), vp['out_w'])


def _final_body(n0_ref, n1_ref, n2_ref, cntg_ref, u_out):
    s = n0_ref[...] + n1_ref[...] + n2_ref[...]
    u_out[...] = s[:, :1] / jnp.maximum(cntg_ref[:, :1], 1.0)


def _call_final(n0, n1, n2, cnt_g):
    return pl.pallas_call(
        _final_body,
        grid=(1,),
        in_specs=[_full((G_, H_))] * 3 + [_full((G_, 16))],
        out_specs=_full((G_, 1)),
        out_shape=jax.ShapeDtypeStruct((G_, 1), _F32),
    )(n0, n1, n2, cnt_g)


def kernel(z, dist, angle, torsion, i, j, idx_kj, idx_ji, batch, params):
    z = z.astype(jnp.int32)
    i = i.astype(jnp.int32)
    j = j.astype(jnp.int32)
    idx_kj = idx_kj.astype(jnp.int32)
    idx_ji = idx_ji.astype(jnp.int32)
    batch = batch.astype(jnp.int32)

    p0 = params['init']
    x = _gather_rows(p0['emb'], z)
    xi = _gather_rows(x, i)
    xj = _gather_rows(x, j)

    rbf0p, e1, e2 = _call_init(dist.reshape(E_, 1), xi, xj, p0)

    cnt_e = _counts(idx_ji, E_)
    cnt_n = _counts(i, N_)
    cnt_g = _counts(batch, G_)

    rbf_kj = _gather_rows(rbf0p, idx_kj)
    ang2 = angle.reshape(T_, 1)
    tor2 = torsion.reshape(T_, 1)
    batch3 = batch.reshape(N_ // BN, 1, BN)

    def vnum(e2_, vp):
        s = _scatter_add(e2_, i, N_)
        return _call_vfn(s, jnp.zeros_like(s), cnt_n, batch3, vp)

    nums = [vnum(e2, params['vs'][0])]
    for lp, vp in zip(params['layers'], params['vs'][1:]):
        xji, xkd = _call_edge1(e1, rbf0p, lp)
        g = _gather_rows(xkd, idx_kj)
        y = _call_trip(g, rbf_kj, ang2, tor2, lp)
        esum = _scatter_add(y, idx_ji, E_)
        e1, e2 = _call_edge2(esum, cnt_e, xji, e1, rbf0p, lp)
        nums.append(vnum(e2, vp))

    return _call_final(nums[0], nums[1], nums[2], cnt_g)
